# baseline (speedup 1.0000x reference)
import functools

import jax
import jax.numpy as jnp
from jax import lax
from jax.experimental import pallas as pl
from jax.experimental.pallas import tpu as pltpu
from jax.experimental.pallas import tpu_sc as plsc

VOCAB = 1000
EMBED_DIM = 64
BATCH = 1024
HIST = 200

NUM_CORES = 2
NUM_SUBCORES = 16
NW = NUM_CORES * NUM_SUBCORES
BATCH_PER_W = BATCH // NW
CHUNK = HIST
HALF = CHUNK // 2
NCHUNK = BATCH_PER_W
NBUF = 4

_mesh = plsc.VectorSubcoreMesh(core_axis_name="c", subcore_axis_name="s")


@functools.partial(
    pl.kernel,
    out_type=jax.ShapeDtypeStruct((BATCH * HIST // 2, 2 * EMBED_DIM), jnp.float32),
    mesh=_mesh,
    scratch_types=[
        pltpu.VMEM((NCHUNK, 2, HALF), jnp.int32),
        pltpu.VMEM((NBUF, 2, HALF, EMBED_DIM), jnp.float32),
        [pltpu.SemaphoreType.DMA] * NBUF,
    ],
    compiler_params=pltpu.CompilerParams(use_tc_tiling_on_sc=False),
)
def _lookup(idx_hbm, table_hbm, out_hbm, idx_v, rows_v, gsems):
    wid = lax.axis_index("s") * NUM_CORES + lax.axis_index("c")
    pltpu.sync_copy(idx_hbm.at[wid], idx_v)

    def start(j, b):
        pltpu.async_copy(table_hbm.at[idx_v.at[j, 0]], rows_v.at[b, 0], gsems[b])
        pltpu.async_copy(table_hbm.at[idx_v.at[j, 1]], rows_v.at[b, 1], gsems[b])

    def finish(j, b):
        pltpu.make_async_copy(
            table_hbm.at[idx_v.at[j, 0]], rows_v.at[b, 0], gsems[b]
        ).wait()
        pltpu.make_async_copy(
            table_hbm.at[idx_v.at[j, 1]], rows_v.at[b, 1], gsems[b]
        ).wait()
        r0 = (wid * BATCH_PER_W + j) * HALF
        pltpu.sync_copy(
            rows_v.at[b, 0], out_hbm.at[pl.ds(r0, HALF), pl.ds(0, EMBED_DIM)]
        )
        pltpu.sync_copy(
            rows_v.at[b, 1],
            out_hbm.at[pl.ds(r0, HALF), pl.ds(EMBED_DIM, EMBED_DIM)],
        )

    for p in range(NBUF - 1):
        start(p, p)

    def outer(i, carry):
        for b in range(NBUF):
            j = i * NBUF + b
            nxt = j + NBUF - 1
            nb = (b + NBUF - 1) % NBUF

            @pl.when(nxt < NCHUNK)
            def _():
                start(nxt, nb)

            finish(j, b)
        return carry

    lax.fori_loop(0, NCHUNK // NBUF, outer, 0)


def kernel(token_indices, embeddings):
    idx = token_indices.reshape(NW, NCHUNK, HALF, 2).transpose(0, 1, 3, 2)
    out = _lookup(idx, embeddings)
    return out.reshape(BATCH, HIST, EMBED_DIM)


# speedup vs baseline: 1.0753x; 1.0753x over previous
import functools

import jax
import jax.numpy as jnp
from jax import lax
from jax.experimental import pallas as pl
from jax.experimental.pallas import tpu as pltpu
from jax.experimental.pallas import tpu_sc as plsc

VOCAB = 1000
EMBED_DIM = 64
BATCH = 1024
HIST = 200

NUM_CORES = 2
NUM_SUBCORES = 16
NW = NUM_CORES * NUM_SUBCORES
NBT = BATCH // 128
H_PER_W = HIST // (NW // NBT)
LANES = 16
NGRP = 128 // LANES

_mesh = plsc.VectorSubcoreMesh(core_axis_name="c", subcore_axis_name="s")


@functools.partial(
    pl.kernel,
    out_type=jax.ShapeDtypeStruct((HIST, 8, NBT, 8, 128), jnp.float32),
    mesh=_mesh,
    scratch_types=[
        pltpu.VMEM((VOCAB * EMBED_DIM,), jnp.float32),
        pltpu.VMEM((H_PER_W, 128), jnp.int32),
        pltpu.VMEM((2, 8, 8, 128), jnp.float32),
        [pltpu.SemaphoreType.DMA] * 2,
    ],
    compiler_params=pltpu.CompilerParams(
        use_tc_tiling_on_sc=False, needs_layout_passes=False
    ),
)
def _lookup(idx_hbm, table_hbm, out_hbm, table_v, idx_v, block_v, wsems):
    wid = lax.axis_index("s") * NUM_CORES + lax.axis_index("c")
    bt = wid % NBT
    h0 = (wid // NBT) * H_PER_W

    pltpu.sync_copy(table_hbm, table_v)
    pltpu.sync_copy(
        idx_hbm.at[pl.ds(h0, H_PER_W), pl.ds(bt * 128, 128)], idx_v
    )

    def compute(u, buf):
        for g in range(NGRP):
            tok = idx_v[u, pl.ds(g * LANES, LANES)]
            base = tok * EMBED_DIM
            for ct in range(8):
                vals = [
                    plsc.load_gather(table_v, [base + (ct * 8 + ci)])
                    for ci in range(8)
                ]
                for ci in range(8):
                    block_v[buf, ct, ci, pl.ds(g * LANES, LANES)] = vals[ci]

    def start_w(u, buf):
        pltpu.async_copy(
            block_v.at[buf], out_hbm.at[h0 + u, :, bt], wsems[buf]
        )

    def wait_w(u, buf):
        pltpu.make_async_copy(
            block_v.at[buf], out_hbm.at[h0 + u, :, bt], wsems[buf]
        ).wait()

    compute(0, 0)
    start_w(0, 0)
    compute(1, 1)
    start_w(1, 1)

    def outer(i, carry):
        for b2 in range(2):
            u = 2 * i + b2
            wait_w(u - 2, b2)
            compute(u, b2)
            start_w(u, b2)
        return carry

    lax.fori_loop(1, H_PER_W // 2, outer, 0)
    wait_w(H_PER_W - 2, 0)
    wait_w(H_PER_W - 1, 1)


def kernel(token_indices, embeddings):
    out5 = _lookup(token_indices.T, embeddings.reshape(VOCAB * EMBED_DIM))
    return out5.transpose(2, 4, 0, 1, 3).reshape(BATCH, HIST, EMBED_DIM)
